# Initial kernel scaffold; baseline (speedup 1.0000x reference)
import jax
import jax.numpy as jnp
from jax.experimental import pallas as pl


def kernel(x, edge_index, W_emb, b_emb, W1, b1, W2, b2, W3, b3):
    raise NotImplementedError("write your pallas kernel here")


# speedup vs baseline: 9.0089x; 9.0089x over previous
import functools

import jax
import jax.numpy as jnp
from jax import lax
from jax.experimental import pallas as pl
from jax.experimental.pallas import tpu as pltpu
from jax.experimental.pallas import tpu_sc as plsc

N = 10000
E = 160000
DH = 256
HALF = DH // 2

NC, NS, L = 2, 16, 16
NW = NC * NS
EPW = E // NW
EPS = E // NS
CHUNK = 128
NFULL = EPS // CHUNK
TAIL = EPS - NFULL * CHUNK
RPT = 632

_MESH = plsc.VectorSubcoreMesh(
    core_axis_name="c", subcore_axis_name="s", num_cores=NC, num_subcores=NS)


def _deg_body(dst_hbm, out_hbm, dbuf, hist):
    c = lax.axis_index("c")
    s = lax.axis_index("s")
    wid = s * NC + c

    def zero(i, _):
        hist[pl.ds(i * L, L)] = jnp.zeros((L,), jnp.float32)
        return 0
    lax.fori_loop(0, N // L, zero, 0)

    pltpu.sync_copy(dst_hbm.at[pl.ds(wid * EPW, EPW)], dbuf.at[pl.ds(0, EPW)])
    ones = jnp.ones((L,), jnp.float32)

    def body(i, _):
        plsc.addupdate_scatter(hist, [dbuf[pl.ds(i * L, L)]], ones)
        return 0
    lax.fori_loop(0, EPW // L, body, 0)
    rem = EPW - (EPW // L) * L
    if rem:
        mask = lax.iota(jnp.int32, L) < rem
        plsc.addupdate_scatter(
            hist, [dbuf[pl.ds((EPW // L) * L, L)]], ones, mask=mask)
    pltpu.sync_copy(hist, out_hbm.at[pl.ds(wid * N, N)])


_SC_PARAMS = pltpu.CompilerParams(needs_layout_passes=False)

_deg_call = pl.kernel(
    _deg_body,
    out_type=jax.ShapeDtypeStruct((NW * N,), jnp.float32),
    mesh=_MESH,
    compiler_params=_SC_PARAMS,
    scratch_types=[
        pltpu.VMEM((EPW + 16,), jnp.int32),
        pltpu.VMEM((N,), jnp.float32),
    ],
)


def _agg_body(src_hbm, dst_hbm, g_hbm, out_hbm,
              srcb, dstb, sidx, rows, srcb_t, dstb_t, sidx_t, rows_t,
              acc, sem):
    c = lax.axis_index("c")
    s = lax.axis_index("s")
    coff = c * N

    rbase = jnp.minimum(s * RPT, N - RPT)
    pltpu.sync_copy(g_hbm.at[pl.ds(coff + rbase, RPT)],
                    acc.at[pl.ds(rbase, RPT)])
    plsc.subcore_barrier()

    def body(j, _):
        off = s * EPS + j * CHUNK
        pltpu.sync_copy(src_hbm.at[pl.ds(off, CHUNK)], srcb)
        pltpu.sync_copy(dst_hbm.at[pl.ds(off, CHUNK)], dstb)
        for v in range(CHUNK // L):
            sidx[pl.ds(v * L, L)] = srcb[pl.ds(v * L, L)] + coff
        pltpu.async_copy(g_hbm.at[sidx], rows, sem).wait()
        pltpu.sync_copy(rows, acc.at[dstb], add=True)
        return 0
    lax.fori_loop(0, NFULL, body, 0)

    if TAIL:
        off = s * EPS + NFULL * CHUNK
        pltpu.sync_copy(src_hbm.at[pl.ds(off, TAIL)], srcb_t)
        pltpu.sync_copy(dst_hbm.at[pl.ds(off, TAIL)], dstb_t)
        sidx_t[...] = srcb_t[...] + coff
        pltpu.async_copy(g_hbm.at[sidx_t], rows_t, sem).wait()
        pltpu.sync_copy(rows_t, acc.at[dstb_t], add=True)

    plsc.subcore_barrier()
    pltpu.sync_copy(acc.at[pl.ds(rbase, RPT)],
                    out_hbm.at[pl.ds(coff + rbase, RPT)])


_agg_call = pl.kernel(
    _agg_body,
    out_type=jax.ShapeDtypeStruct((NC * N, HALF), jnp.float32),
    mesh=_MESH,
    compiler_params=_SC_PARAMS,
    scratch_types=[
        pltpu.VMEM((CHUNK,), jnp.int32),
        pltpu.VMEM((CHUNK,), jnp.int32),
        pltpu.VMEM((CHUNK,), jnp.int32),
        pltpu.VMEM((CHUNK, HALF), jnp.float32),
        pltpu.VMEM((TAIL,), jnp.int32),
        pltpu.VMEM((TAIL,), jnp.int32),
        pltpu.VMEM((TAIL,), jnp.int32),
        pltpu.VMEM((TAIL, HALF), jnp.float32),
        pltpu.VMEM_SHARED((N, HALF), jnp.float32),
        pltpu.SemaphoreType.DMA,
    ],
)


def _prep_body(x_ref, w_ref, b_ref, dp_ref, h_ref, dinv_ref):
    h_ref[...] = jnp.dot(x_ref[...], w_ref[...],
                         preferred_element_type=jnp.float32) + b_ref[...]
    deg = jnp.sum(dp_ref[...], axis=0) + 1.0
    dinv_ref[...] = lax.rsqrt(deg)[:, None]


def _prep_call(x, w_emb, b_emb, degp):
    return pl.pallas_call(
        _prep_body,
        out_shape=[
            jax.ShapeDtypeStruct((N, DH), jnp.float32),
            jax.ShapeDtypeStruct((N, 1), jnp.float32),
        ],
    )(x, w_emb, b_emb, degp)


def _mm_body(h_ref, w_ref, dinv_ref, g_ref):
    t = jnp.dot(h_ref[...], w_ref[...],
                preferred_element_type=jnp.float32) * dinv_ref[...]
    g_ref[0] = t[:, :HALF]
    g_ref[1] = t[:, HALF:]


def _mm_call(h, w, dinv):
    blk = 1000
    return pl.pallas_call(
        _mm_body,
        grid=(N // blk,),
        in_specs=[
            pl.BlockSpec((blk, DH), lambda i: (i, 0)),
            pl.BlockSpec((DH, DH), lambda i: (0, 0)),
            pl.BlockSpec((blk, 1), lambda i: (i, 0)),
        ],
        out_specs=pl.BlockSpec((NC, blk, HALF), lambda i: (0, i, 0)),
        out_shape=jax.ShapeDtypeStruct((NC, N, HALF), jnp.float32),
    )(h, w, dinv)


def _post_body(acc_ref, h_ref, dinv_ref, b_ref, out_ref):
    a = jnp.concatenate([acc_ref[0], acc_ref[1]], axis=1)
    out_ref[...] = jax.nn.relu(a * dinv_ref[...] + b_ref[...]) + h_ref[...]


def _post_call(acc, h, dinv, b):
    blk = 1000
    return pl.pallas_call(
        _post_body,
        grid=(N // blk,),
        in_specs=[
            pl.BlockSpec((NC, blk, HALF), lambda i: (0, i, 0)),
            pl.BlockSpec((blk, DH), lambda i: (i, 0)),
            pl.BlockSpec((blk, 1), lambda i: (i, 0)),
            pl.BlockSpec((1, DH), lambda i: (0, 0)),
        ],
        out_specs=pl.BlockSpec((blk, DH), lambda i: (i, 0)),
        out_shape=jax.ShapeDtypeStruct((N, DH), jnp.float32),
    )(acc, h, dinv, b)


def kernel(x, edge_index, W_emb, b_emb, W1, b1, W2, b2, W3, b3):
    src = edge_index[0]
    dst = edge_index[1]
    degp = _deg_call(dst).reshape(NW, N)
    h, dinv = _prep_call(x, W_emb, b_emb.reshape(1, DH), degp)
    for W, b in ((W1, b1), (W2, b2), (W3, b3)):
        g2 = _mm_call(h, W, dinv)
        accf = _agg_call(src, dst, g2.reshape(NC * N, HALF))
        h = _post_call(accf.reshape(NC, N, HALF), h, dinv, b.reshape(1, DH))
    return h


# speedup vs baseline: 13.7244x; 1.5234x over previous
import functools

import jax
import jax.numpy as jnp
from jax import lax
from jax.experimental import pallas as pl
from jax.experimental.pallas import tpu as pltpu
from jax.experimental.pallas import tpu_sc as plsc

N = 10000
E = 160000
DH = 256
HALF = DH // 2

NC, NS, L = 2, 16, 16
NW = NC * NS
EPW = E // NW
EPS = E // NS
CHUNK = 128
NFULL = EPS // CHUNK
TAIL = EPS - NFULL * CHUNK
RPT = 632

_MESH = plsc.VectorSubcoreMesh(
    core_axis_name="c", subcore_axis_name="s", num_cores=NC, num_subcores=NS)


def _deg_body(dst_hbm, out_hbm, dbuf, hist):
    c = lax.axis_index("c")
    s = lax.axis_index("s")
    wid = s * NC + c

    def zero(i, _):
        hist[pl.ds(i * L, L)] = jnp.zeros((L,), jnp.float32)
        return 0
    lax.fori_loop(0, N // L, zero, 0)

    pltpu.sync_copy(dst_hbm.at[pl.ds(wid * EPW, EPW)], dbuf.at[pl.ds(0, EPW)])
    ones = jnp.ones((L,), jnp.float32)

    def body(i, _):
        plsc.addupdate_scatter(hist, [dbuf[pl.ds(i * L, L)]], ones)
        return 0
    lax.fori_loop(0, EPW // L, body, 0)
    rem = EPW - (EPW // L) * L
    if rem:
        mask = lax.iota(jnp.int32, L) < rem
        plsc.addupdate_scatter(
            hist, [dbuf[pl.ds((EPW // L) * L, L)]], ones, mask=mask)
    pltpu.sync_copy(hist, out_hbm.at[pl.ds(wid * N, N)])


_SC_PARAMS = pltpu.CompilerParams(needs_layout_passes=False)

_deg_call = pl.kernel(
    _deg_body,
    out_type=jax.ShapeDtypeStruct((NW * N,), jnp.float32),
    mesh=_MESH,
    compiler_params=_SC_PARAMS,
    scratch_types=[
        pltpu.VMEM((EPW + 16,), jnp.int32),
        pltpu.VMEM((N,), jnp.float32),
    ],
)


def _agg_body(src_hbm, dst_hbm, g_hbm, out_hbm,
              srcb_all, dstc0, dstc1, rows0, rows1,
              srcb_t, dstb_t, sidx_t, rows_t,
              acc, gsem0, gsem1, ssem0, ssem1, dsem0, dsem1, tsem):
    c = lax.axis_index("c")
    s = lax.axis_index("s")
    coff = c * N
    ebase = s * EPS

    idx_cp = pltpu.async_copy(src_hbm.at[pl.ds(ebase, EPS)], srcb_all, gsem0)

    rbase = jnp.minimum(s * RPT, N - RPT)
    pltpu.sync_copy(g_hbm.at[pl.ds(coff + rbase, RPT)],
                    acc.at[pl.ds(rbase, RPT)])
    idx_cp.wait()

    def shift(v, _):
        sl = pl.ds(v * L, L)
        srcb_all[sl] = srcb_all[sl] + coff
        return 0
    lax.fori_loop(0, EPS // L, shift, 0)
    plsc.subcore_barrier()

    dstcs = (dstc0, dstc1)
    rowss = (rows0, rows1)
    gsems = (gsem0, gsem1)
    ssems = (ssem0, ssem1)
    dsems = (dsem0, dsem1)

    def start_chunk(j, b):
        pltpu.async_copy(dst_hbm.at[pl.ds(ebase + j * CHUNK, CHUNK)],
                         dstcs[b], dsems[b])
        pltpu.async_copy(g_hbm.at[srcb_all.at[pl.ds(j * CHUNK, CHUNK)]],
                         rowss[b], gsems[b])

    def wait_chunk(j, b):
        pltpu.make_async_copy(dst_hbm.at[pl.ds(ebase + j * CHUNK, CHUNK)],
                              dstcs[b], dsems[b]).wait()
        pltpu.make_async_copy(g_hbm.at[srcb_all.at[pl.ds(j * CHUNK, CHUNK)]],
                              rowss[b], gsems[b]).wait()

    def start_scatter(b):
        pltpu.async_copy(rowss[b], acc.at[dstcs[b]], ssems[b], add=True)

    def wait_scatter(b):
        pltpu.make_async_copy(rowss[b], acc.at[dstcs[b]], ssems[b]).wait()

    start_chunk(0, 0)
    start_chunk(1, 1)

    def body(i, _):
        wait_chunk(2 * i, 0)
        start_scatter(0)
        wait_chunk(2 * i + 1, 1)
        start_scatter(1)

        @pl.when(i < NFULL // 2 - 1)
        def _():
            wait_scatter(0)
            start_chunk(2 * i + 2, 0)
            wait_scatter(1)
            start_chunk(2 * i + 3, 1)
        return 0
    lax.fori_loop(0, NFULL // 2, body, 0)
    wait_scatter(0)
    wait_scatter(1)

    if TAIL:
        off = ebase + NFULL * CHUNK
        pltpu.sync_copy(src_hbm.at[pl.ds(off, TAIL)], srcb_t)
        pltpu.sync_copy(dst_hbm.at[pl.ds(off, TAIL)], dstb_t)
        sidx_t[...] = srcb_t[...] + coff
        pltpu.async_copy(g_hbm.at[sidx_t], rows_t, tsem).wait()
        pltpu.sync_copy(rows_t, acc.at[dstb_t], add=True)

    plsc.subcore_barrier()
    pltpu.sync_copy(acc.at[pl.ds(rbase, RPT)],
                    out_hbm.at[pl.ds(coff + rbase, RPT)])


_agg_call = pl.kernel(
    _agg_body,
    out_type=jax.ShapeDtypeStruct((NC * N, HALF), jnp.float32),
    mesh=_MESH,
    compiler_params=_SC_PARAMS,
    scratch_types=[
        pltpu.VMEM((EPS,), jnp.int32),
        pltpu.VMEM((CHUNK,), jnp.int32),
        pltpu.VMEM((CHUNK,), jnp.int32),
        pltpu.VMEM((CHUNK, HALF), jnp.float32),
        pltpu.VMEM((CHUNK, HALF), jnp.float32),
        pltpu.VMEM((TAIL,), jnp.int32),
        pltpu.VMEM((TAIL,), jnp.int32),
        pltpu.VMEM((TAIL,), jnp.int32),
        pltpu.VMEM((TAIL, HALF), jnp.float32),
        pltpu.VMEM_SHARED((N, HALF), jnp.float32),
        pltpu.SemaphoreType.DMA,
        pltpu.SemaphoreType.DMA,
        pltpu.SemaphoreType.DMA,
        pltpu.SemaphoreType.DMA,
        pltpu.SemaphoreType.DMA,
        pltpu.SemaphoreType.DMA,
        pltpu.SemaphoreType.DMA,
    ],
)


def _prep_body(x_ref, w_ref, b_ref, dp_ref, h_ref, dinv_ref):
    h_ref[...] = jnp.dot(x_ref[...], w_ref[...],
                         preferred_element_type=jnp.float32) + b_ref[...]
    deg = jnp.sum(dp_ref[...], axis=0) + 1.0
    dinv_ref[...] = lax.rsqrt(deg)[:, None]


def _prep_call(x, w_emb, b_emb, degp):
    return pl.pallas_call(
        _prep_body,
        out_shape=[
            jax.ShapeDtypeStruct((N, DH), jnp.float32),
            jax.ShapeDtypeStruct((N, 1), jnp.float32),
        ],
    )(x, w_emb, b_emb, degp)


def _mm_body(h_ref, w_ref, dinv_ref, g_ref):
    t = jnp.dot(h_ref[...], w_ref[...],
                preferred_element_type=jnp.float32) * dinv_ref[...]
    g_ref[0] = t[:, :HALF]
    g_ref[1] = t[:, HALF:]


def _mm_call(h, w, dinv):
    blk = 1000
    return pl.pallas_call(
        _mm_body,
        grid=(N // blk,),
        in_specs=[
            pl.BlockSpec((blk, DH), lambda i: (i, 0)),
            pl.BlockSpec((DH, DH), lambda i: (0, 0)),
            pl.BlockSpec((blk, 1), lambda i: (i, 0)),
        ],
        out_specs=pl.BlockSpec((NC, blk, HALF), lambda i: (0, i, 0)),
        out_shape=jax.ShapeDtypeStruct((NC, N, HALF), jnp.float32),
    )(h, w, dinv)


def _post_body(acc_ref, h_ref, dinv_ref, b_ref, out_ref):
    a = jnp.concatenate([acc_ref[0], acc_ref[1]], axis=1)
    out_ref[...] = jax.nn.relu(a * dinv_ref[...] + b_ref[...]) + h_ref[...]


def _post_call(acc, h, dinv, b):
    blk = 1000
    return pl.pallas_call(
        _post_body,
        grid=(N // blk,),
        in_specs=[
            pl.BlockSpec((NC, blk, HALF), lambda i: (0, i, 0)),
            pl.BlockSpec((blk, DH), lambda i: (i, 0)),
            pl.BlockSpec((blk, 1), lambda i: (i, 0)),
            pl.BlockSpec((1, DH), lambda i: (0, 0)),
        ],
        out_specs=pl.BlockSpec((blk, DH), lambda i: (i, 0)),
        out_shape=jax.ShapeDtypeStruct((N, DH), jnp.float32),
    )(acc, h, dinv, b)


def kernel(x, edge_index, W_emb, b_emb, W1, b1, W2, b2, W3, b3):
    src = edge_index[0]
    dst = edge_index[1]
    degp = _deg_call(dst).reshape(NW, N)
    h, dinv = _prep_call(x, W_emb, b_emb.reshape(1, DH), degp)
    for W, b in ((W1, b1), (W2, b2), (W3, b3)):
        g2 = _mm_call(h, W, dinv)
        accf = _agg_call(src, dst, g2.reshape(NC * N, HALF))
        h = _post_call(accf.reshape(NC, N, HALF), h, dinv, b.reshape(1, DH))
    return h


# speedup vs baseline: 18.5274x; 1.3500x over previous
import functools

import jax
import jax.numpy as jnp
from jax import lax
from jax.experimental import pallas as pl
from jax.experimental.pallas import tpu as pltpu
from jax.experimental.pallas import tpu_sc as plsc

N = 10000
E = 160000
DH = 256
HALF = DH // 2

NC, NS, L = 2, 16, 16
NW = NC * NS
EPW = E // NW
EPS = E // NS
CHUNK = 80
NCH = EPS // CHUNK
NSLOT = 3
RPT = 632

_MESH = plsc.VectorSubcoreMesh(
    core_axis_name="c", subcore_axis_name="s", num_cores=NC, num_subcores=NS)


def _deg_body(dst_hbm, out_hbm, dbuf, hist):
    c = lax.axis_index("c")
    s = lax.axis_index("s")
    wid = s * NC + c

    def zero(i, _):
        hist[pl.ds(i * L, L)] = jnp.zeros((L,), jnp.float32)
        return 0
    lax.fori_loop(0, N // L, zero, 0)

    pltpu.sync_copy(dst_hbm.at[pl.ds(wid * EPW, EPW)], dbuf.at[pl.ds(0, EPW)])
    ones = jnp.ones((L,), jnp.float32)

    def body(i, _):
        plsc.addupdate_scatter(hist, [dbuf[pl.ds(i * L, L)]], ones)
        return 0
    lax.fori_loop(0, EPW // L, body, 0)
    rem = EPW - (EPW // L) * L
    if rem:
        mask = lax.iota(jnp.int32, L) < rem
        plsc.addupdate_scatter(
            hist, [dbuf[pl.ds((EPW // L) * L, L)]], ones, mask=mask)
    pltpu.sync_copy(hist, out_hbm.at[pl.ds(wid * N, N)])


_SC_PARAMS = pltpu.CompilerParams(needs_layout_passes=False)

_deg_call = pl.kernel(
    _deg_body,
    out_type=jax.ShapeDtypeStruct((NW * N,), jnp.float32),
    mesh=_MESH,
    compiler_params=_SC_PARAMS,
    scratch_types=[
        pltpu.VMEM((EPW + 16,), jnp.int32),
        pltpu.VMEM((N,), jnp.float32),
    ],
)


def _agg_body(src_hbm, dst_hbm, g_hbm, out_hbm,
              srcb_all, dstc0, dstc1, dstc2, rows0, rows1, rows2,
              acc, isem, gsem0, gsem1, gsem2, ssem0, ssem1, ssem2,
              dsem0, dsem1, dsem2):
    c = lax.axis_index("c")
    s = lax.axis_index("s")
    coff = c * N
    ebase = s * EPS

    idx_cp = pltpu.async_copy(src_hbm.at[pl.ds(ebase, EPS)], srcb_all, isem)

    rbase = jnp.minimum(s * RPT, N - RPT)
    pltpu.sync_copy(g_hbm.at[pl.ds(coff + rbase, RPT)],
                    acc.at[pl.ds(rbase, RPT)])
    idx_cp.wait()

    def shift(v, _):
        sl = pl.ds(v * L, L)
        srcb_all[sl] = srcb_all[sl] + coff
        return 0
    lax.fori_loop(0, EPS // L, shift, 0)
    plsc.subcore_barrier()

    dstcs = (dstc0, dstc1, dstc2)
    rowss = (rows0, rows1, rows2)
    gsems = (gsem0, gsem1, gsem2)
    ssems = (ssem0, ssem1, ssem2)
    dsems = (dsem0, dsem1, dsem2)

    def start_chunk(j, b):
        pltpu.async_copy(dst_hbm.at[pl.ds(ebase + j * CHUNK, CHUNK)],
                         dstcs[b], dsems[b])
        pltpu.async_copy(g_hbm.at[srcb_all.at[pl.ds(j * CHUNK, CHUNK)]],
                         rowss[b], gsems[b])

    def wait_chunk(j, b):
        pltpu.make_async_copy(dst_hbm.at[pl.ds(ebase + j * CHUNK, CHUNK)],
                              dstcs[b], dsems[b]).wait()
        pltpu.make_async_copy(g_hbm.at[srcb_all.at[pl.ds(j * CHUNK, CHUNK)]],
                              rowss[b], gsems[b]).wait()

    def start_scatter(b):
        pltpu.async_copy(rowss[b], acc.at[dstcs[b]], ssems[b], add=True)

    def wait_scatter(b):
        pltpu.make_async_copy(rowss[b], acc.at[dstcs[b]], ssems[b]).wait()

    for b in range(NSLOT):
        start_chunk(b, b)

    def body(i, _):
        for b in range(NSLOT):
            j = NSLOT * i + b

            @pl.when(j < NCH)
            def _():
                wait_chunk(j, b)
                start_scatter(b)

            @pl.when(j + NSLOT < NCH)
            def _():
                wait_scatter(b)
                start_chunk(j + NSLOT, b)
        return 0
    lax.fori_loop(0, (NCH + NSLOT - 1) // NSLOT, body, 0)
    for b in range(NSLOT):
        wait_scatter(b)

    plsc.subcore_barrier()
    pltpu.sync_copy(acc.at[pl.ds(rbase, RPT)],
                    out_hbm.at[pl.ds(coff + rbase, RPT)])


_agg_call = pl.kernel(
    _agg_body,
    out_type=jax.ShapeDtypeStruct((NC * N, HALF), jnp.float32),
    mesh=_MESH,
    compiler_params=_SC_PARAMS,
    scratch_types=(
        [pltpu.VMEM((EPS,), jnp.int32)]
        + [pltpu.VMEM((CHUNK,), jnp.int32) for _ in range(NSLOT)]
        + [pltpu.VMEM((CHUNK, HALF), jnp.float32) for _ in range(NSLOT)]
        + [pltpu.VMEM_SHARED((N, HALF), jnp.float32)]
        + [pltpu.SemaphoreType.DMA for _ in range(1 + 3 * NSLOT)]
    ),
)


def _prep_body(x_ref, w_ref, b_ref, dp_ref, h_ref, dinv_ref):
    h_ref[...] = jnp.dot(x_ref[...], w_ref[...],
                         preferred_element_type=jnp.float32) + b_ref[...]
    deg = jnp.sum(dp_ref[...], axis=0) + 1.0
    dinv_ref[...] = lax.rsqrt(deg)[:, None]


def _prep_call(x, w_emb, b_emb, degp):
    return pl.pallas_call(
        _prep_body,
        out_shape=[
            jax.ShapeDtypeStruct((N, DH), jnp.float32),
            jax.ShapeDtypeStruct((N, 1), jnp.float32),
        ],
    )(x, w_emb, b_emb, degp)


def _mm_body(h_ref, w_ref, dinv_ref, g_ref):
    t = jnp.dot(h_ref[...], w_ref[...],
                preferred_element_type=jnp.float32) * dinv_ref[...]
    g_ref[0] = t[:, :HALF]
    g_ref[1] = t[:, HALF:]


def _mm_call(h, w, dinv):
    blk = 1000
    return pl.pallas_call(
        _mm_body,
        grid=(N // blk,),
        in_specs=[
            pl.BlockSpec((blk, DH), lambda i: (i, 0)),
            pl.BlockSpec((DH, DH), lambda i: (0, 0)),
            pl.BlockSpec((blk, 1), lambda i: (i, 0)),
        ],
        out_specs=pl.BlockSpec((NC, blk, HALF), lambda i: (0, i, 0)),
        out_shape=jax.ShapeDtypeStruct((NC, N, HALF), jnp.float32),
    )(h, w, dinv)


def _post_body(acc_ref, h_ref, dinv_ref, b_ref, out_ref):
    a = jnp.concatenate([acc_ref[0], acc_ref[1]], axis=1)
    out_ref[...] = jax.nn.relu(a * dinv_ref[...] + b_ref[...]) + h_ref[...]


def _post_call(acc, h, dinv, b):
    blk = 1000
    return pl.pallas_call(
        _post_body,
        grid=(N // blk,),
        in_specs=[
            pl.BlockSpec((NC, blk, HALF), lambda i: (0, i, 0)),
            pl.BlockSpec((blk, DH), lambda i: (i, 0)),
            pl.BlockSpec((blk, 1), lambda i: (i, 0)),
            pl.BlockSpec((1, DH), lambda i: (0, 0)),
        ],
        out_specs=pl.BlockSpec((blk, DH), lambda i: (i, 0)),
        out_shape=jax.ShapeDtypeStruct((N, DH), jnp.float32),
    )(acc, h, dinv, b)


def kernel(x, edge_index, W_emb, b_emb, W1, b1, W2, b2, W3, b3):
    src = edge_index[0]
    dst = edge_index[1]
    degp = _deg_call(dst).reshape(NW, N)
    h, dinv = _prep_call(x, W_emb, b_emb.reshape(1, DH), degp)
    for W, b in ((W1, b1), (W2, b2), (W3, b3)):
        g2 = _mm_call(h, W, dinv)
        accf = _agg_call(src, dst, g2.reshape(NC * N, HALF))
        h = _post_call(accf.reshape(NC, N, HALF), h, dinv, b.reshape(1, DH))
    return h


# speedup vs baseline: 19.8543x; 1.0716x over previous
import functools

import jax
import jax.numpy as jnp
from jax import lax
from jax.experimental import pallas as pl
from jax.experimental.pallas import tpu as pltpu
from jax.experimental.pallas import tpu_sc as plsc

N = 10000
E = 160000
DH = 256
HALF = DH // 2

NC, NS, L = 2, 16, 16
NW = NC * NS
EPW = E // NW
EPS = E // NS
CHUNK = 80
NCH = EPS // CHUNK
NSLOT = 3
RPT = 632

_MESH = plsc.VectorSubcoreMesh(
    core_axis_name="c", subcore_axis_name="s", num_cores=NC, num_subcores=NS)


def _deg_body(dst_hbm, out_hbm, dbuf, hist):
    c = lax.axis_index("c")
    s = lax.axis_index("s")
    wid = s * NC + c

    def zero(i, _):
        hist[pl.ds(i * L, L)] = jnp.zeros((L,), jnp.float32)
        return 0
    lax.fori_loop(0, N // L, zero, 0)

    pltpu.sync_copy(dst_hbm.at[pl.ds(wid * EPW, EPW)], dbuf.at[pl.ds(0, EPW)])
    ones = jnp.ones((L,), jnp.float32)

    def body(i, _):
        plsc.addupdate_scatter(hist, [dbuf[pl.ds(i * L, L)]], ones)
        return 0
    lax.fori_loop(0, EPW // L, body, 0)
    rem = EPW - (EPW // L) * L
    if rem:
        mask = lax.iota(jnp.int32, L) < rem
        plsc.addupdate_scatter(
            hist, [dbuf[pl.ds((EPW // L) * L, L)]], ones, mask=mask)
    pltpu.sync_copy(hist, out_hbm.at[pl.ds(wid * N, N)])


_SC_PARAMS = pltpu.CompilerParams(needs_layout_passes=False)

_deg_call = pl.kernel(
    _deg_body,
    out_type=jax.ShapeDtypeStruct((NW * N,), jnp.float32),
    mesh=_MESH,
    compiler_params=_SC_PARAMS,
    scratch_types=[
        pltpu.VMEM((EPW + 16,), jnp.int32),
        pltpu.VMEM((N,), jnp.float32),
    ],
)


def _agg_body(src_hbm, dst_hbm, g_hbm, out_hbm,
              srcb_all, dstc0, dstc1, dstc2, rows0, rows1, rows2,
              acc, isem, gsem0, gsem1, gsem2, ssem0, ssem1, ssem2,
              dsem0, dsem1, dsem2):
    c = lax.axis_index("c")
    s = lax.axis_index("s")
    coff = c * N
    ebase = s * EPS

    idx_cp = pltpu.async_copy(src_hbm.at[pl.ds(ebase, EPS)], srcb_all, isem)

    rbase = jnp.minimum(s * RPT, N - RPT)
    pltpu.sync_copy(g_hbm.at[pl.ds(coff + rbase, RPT)],
                    acc.at[pl.ds(rbase, RPT)])
    idx_cp.wait()

    def shift(v, _):
        sl = pl.ds(v * L, L)
        srcb_all[sl] = srcb_all[sl] + coff
        return 0
    lax.fori_loop(0, EPS // L, shift, 0)
    plsc.subcore_barrier()

    dstcs = (dstc0, dstc1, dstc2)
    rowss = (rows0, rows1, rows2)
    gsems = (gsem0, gsem1, gsem2)
    ssems = (ssem0, ssem1, ssem2)
    dsems = (dsem0, dsem1, dsem2)

    def start_chunk(j, b):
        pltpu.async_copy(dst_hbm.at[pl.ds(ebase + j * CHUNK, CHUNK)],
                         dstcs[b], dsems[b])
        pltpu.async_copy(g_hbm.at[srcb_all.at[pl.ds(j * CHUNK, CHUNK)]],
                         rowss[b], gsems[b])

    def wait_chunk(j, b):
        pltpu.make_async_copy(dst_hbm.at[pl.ds(ebase + j * CHUNK, CHUNK)],
                              dstcs[b], dsems[b]).wait()
        pltpu.make_async_copy(g_hbm.at[srcb_all.at[pl.ds(j * CHUNK, CHUNK)]],
                              rowss[b], gsems[b]).wait()

    def start_scatter(b):
        pltpu.async_copy(rowss[b], acc.at[dstcs[b]], ssems[b], add=True)

    def wait_scatter(b):
        pltpu.make_async_copy(rowss[b], acc.at[dstcs[b]], ssems[b]).wait()

    for b in range(NSLOT):
        start_chunk(b, b)

    def body(i, _):
        for b in range(NSLOT):
            j = NSLOT * i + b

            @pl.when(j < NCH)
            def _():
                wait_chunk(j, b)
                start_scatter(b)

            @pl.when(j + NSLOT < NCH)
            def _():
                wait_scatter(b)
                start_chunk(j + NSLOT, b)
        return 0
    lax.fori_loop(0, (NCH + NSLOT - 1) // NSLOT, body, 0)
    for b in range(NSLOT):
        wait_scatter(b)

    plsc.subcore_barrier()
    pltpu.sync_copy(acc.at[pl.ds(rbase, RPT)],
                    out_hbm.at[pl.ds(coff + rbase, RPT)])


_agg_call = pl.kernel(
    _agg_body,
    out_type=jax.ShapeDtypeStruct((NC * N, HALF), jnp.float32),
    mesh=_MESH,
    compiler_params=_SC_PARAMS,
    scratch_types=(
        [pltpu.VMEM((EPS,), jnp.int32)]
        + [pltpu.VMEM((CHUNK,), jnp.int32) for _ in range(NSLOT)]
        + [pltpu.VMEM((CHUNK, HALF), jnp.float32) for _ in range(NSLOT)]
        + [pltpu.VMEM_SHARED((N, HALF), jnp.float32)]
        + [pltpu.SemaphoreType.DMA for _ in range(1 + 3 * NSLOT)]
    ),
)


def _prep_body(x_ref, we_ref, be_ref, dp_ref, w1_ref, h_ref, dinv_ref, g_ref):
    h = jnp.dot(x_ref[...], we_ref[...],
                preferred_element_type=jnp.float32) + be_ref[...]
    h_ref[...] = h
    deg = jnp.sum(dp_ref[...], axis=0) + 1.0
    dinv = lax.rsqrt(deg)[:, None]
    dinv_ref[...] = dinv
    t = jnp.dot(h, w1_ref[...], preferred_element_type=jnp.float32) * dinv
    g_ref[0] = t[:, :HALF]
    g_ref[1] = t[:, HALF:]


def _prep_call(x, w_emb, b_emb, degp, w1):
    return pl.pallas_call(
        _prep_body,
        out_shape=[
            jax.ShapeDtypeStruct((N, DH), jnp.float32),
            jax.ShapeDtypeStruct((N, 1), jnp.float32),
            jax.ShapeDtypeStruct((NC, N, HALF), jnp.float32),
        ],
    )(x, w_emb, b_emb, degp, w1)


def _fused_body(acc_ref, h_ref, dinv_ref, b_ref, w_ref, hn_ref, g_ref):
    a = jnp.concatenate([acc_ref[0], acc_ref[1]], axis=1)
    hn = jax.nn.relu(a * dinv_ref[...] + b_ref[...]) + h_ref[...]
    hn_ref[...] = hn
    t = jnp.dot(hn, w_ref[...],
                preferred_element_type=jnp.float32) * dinv_ref[...]
    g_ref[0] = t[:, :HALF]
    g_ref[1] = t[:, HALF:]


def _fused_call(acc, h, dinv, b, w):
    blk = 1000
    return pl.pallas_call(
        _fused_body,
        grid=(N // blk,),
        in_specs=[
            pl.BlockSpec((NC, blk, HALF), lambda i: (0, i, 0)),
            pl.BlockSpec((blk, DH), lambda i: (i, 0)),
            pl.BlockSpec((blk, 1), lambda i: (i, 0)),
            pl.BlockSpec((1, DH), lambda i: (0, 0)),
            pl.BlockSpec((DH, DH), lambda i: (0, 0)),
        ],
        out_specs=[
            pl.BlockSpec((blk, DH), lambda i: (i, 0)),
            pl.BlockSpec((NC, blk, HALF), lambda i: (0, i, 0)),
        ],
        out_shape=[
            jax.ShapeDtypeStruct((N, DH), jnp.float32),
            jax.ShapeDtypeStruct((NC, N, HALF), jnp.float32),
        ],
    )(acc, h, dinv, b, w)


def _post_body(acc_ref, h_ref, dinv_ref, b_ref, out_ref):
    a = jnp.concatenate([acc_ref[0], acc_ref[1]], axis=1)
    out_ref[...] = jax.nn.relu(a * dinv_ref[...] + b_ref[...]) + h_ref[...]


def _post_call(acc, h, dinv, b):
    blk = 1000
    return pl.pallas_call(
        _post_body,
        grid=(N // blk,),
        in_specs=[
            pl.BlockSpec((NC, blk, HALF), lambda i: (0, i, 0)),
            pl.BlockSpec((blk, DH), lambda i: (i, 0)),
            pl.BlockSpec((blk, 1), lambda i: (i, 0)),
            pl.BlockSpec((1, DH), lambda i: (0, 0)),
        ],
        out_specs=pl.BlockSpec((blk, DH), lambda i: (i, 0)),
        out_shape=jax.ShapeDtypeStruct((N, DH), jnp.float32),
    )(acc, h, dinv, b)


def kernel(x, edge_index, W_emb, b_emb, W1, b1, W2, b2, W3, b3):
    src = edge_index[0]
    dst = edge_index[1]
    degp = _deg_call(dst).reshape(NW, N)
    h, dinv, g = _prep_call(x, W_emb, b_emb.reshape(1, DH), degp, W1)
    for b, W_next in ((b1, W2), (b2, W3)):
        accf = _agg_call(src, dst, g.reshape(NC * N, HALF))
        h, g = _fused_call(accf.reshape(NC, N, HALF), h, dinv,
                           b.reshape(1, DH), W_next)
    accf = _agg_call(src, dst, g.reshape(NC * N, HALF))
    return _post_call(accf.reshape(NC, N, HALF), h, dinv, b3.reshape(1, DH))


# speedup vs baseline: 20.3843x; 1.0267x over previous
import functools

import jax
import jax.numpy as jnp
from jax import lax
from jax.experimental import pallas as pl
from jax.experimental.pallas import tpu as pltpu
from jax.experimental.pallas import tpu_sc as plsc

N = 10000
E = 160000
DH = 256
HALF = DH // 2

NC, NS, L = 2, 16, 16
NW = NC * NS
EPW = E // NW
EPS = E // NS
CHUNK = 80
NCH = EPS // CHUNK
NSLOT = 3
RPT = 632

_MESH = plsc.VectorSubcoreMesh(
    core_axis_name="c", subcore_axis_name="s", num_cores=NC, num_subcores=NS)


def _deg_body(ei_hbm, out_hbm, dbuf, hist):
    c = lax.axis_index("c")
    s = lax.axis_index("s")
    wid = s * NC + c

    def zero(i, _):
        hist[0, pl.ds(i * L, L)] = jnp.zeros((L,), jnp.float32)
        return 0
    lax.fori_loop(0, N // L, zero, 0)

    pltpu.sync_copy(ei_hbm.at[pl.ds(E + wid * EPW, EPW)],
                    dbuf.at[pl.ds(0, EPW)])
    ones = jnp.ones((L,), jnp.float32)
    hrow = hist.at[0]

    def body(i, _):
        plsc.addupdate_scatter(hrow, [dbuf[pl.ds(i * L, L)]], ones)
        return 0
    lax.fori_loop(0, EPW // L, body, 0)
    rem = EPW - (EPW // L) * L
    if rem:
        mask = lax.iota(jnp.int32, L) < rem
        plsc.addupdate_scatter(
            hrow, [dbuf[pl.ds((EPW // L) * L, L)]], ones, mask=mask)
    pltpu.sync_copy(hist, out_hbm.at[wid])


_SC_PARAMS = pltpu.CompilerParams(needs_layout_passes=False)

_deg_call = pl.kernel(
    _deg_body,
    out_type=jax.ShapeDtypeStruct((NW, 1, N), jnp.float32),
    mesh=_MESH,
    compiler_params=_SC_PARAMS,
    scratch_types=[
        pltpu.VMEM((EPW + 16,), jnp.int32),
        pltpu.VMEM((1, N), jnp.float32),
    ],
)


def _agg_body(ei_hbm, g_hbm, out_hbm,
              srcb_all, dstc0, dstc1, dstc2, rows0, rows1, rows2,
              acc, isem, gsem0, gsem1, gsem2, ssem0, ssem1, ssem2,
              dsem0, dsem1, dsem2):
    c = lax.axis_index("c")
    s = lax.axis_index("s")
    coff = c * N
    ebase = s * EPS

    idx_cp = pltpu.async_copy(ei_hbm.at[pl.ds(ebase, EPS)], srcb_all, isem)

    rbase = jnp.minimum(s * RPT, N - RPT)
    pltpu.sync_copy(g_hbm.at[pl.ds(coff + rbase, RPT)],
                    acc.at[pl.ds(rbase, RPT)])
    idx_cp.wait()

    def shift(v, _):
        sl = pl.ds(v * L, L)
        srcb_all[sl] = srcb_all[sl] + coff
        return 0
    lax.fori_loop(0, EPS // L, shift, 0)
    plsc.subcore_barrier()

    dstcs = (dstc0, dstc1, dstc2)
    rowss = (rows0, rows1, rows2)
    gsems = (gsem0, gsem1, gsem2)
    ssems = (ssem0, ssem1, ssem2)
    dsems = (dsem0, dsem1, dsem2)

    def start_chunk(j, b):
        pltpu.async_copy(ei_hbm.at[pl.ds(E + ebase + j * CHUNK, CHUNK)],
                         dstcs[b], dsems[b])
        pltpu.async_copy(g_hbm.at[srcb_all.at[pl.ds(j * CHUNK, CHUNK)]],
                         rowss[b], gsems[b])

    def wait_chunk(j, b):
        pltpu.make_async_copy(ei_hbm.at[pl.ds(E + ebase + j * CHUNK, CHUNK)],
                              dstcs[b], dsems[b]).wait()
        pltpu.make_async_copy(g_hbm.at[srcb_all.at[pl.ds(j * CHUNK, CHUNK)]],
                              rowss[b], gsems[b]).wait()

    def start_scatter(b):
        pltpu.async_copy(rowss[b], acc.at[dstcs[b]], ssems[b], add=True)

    def wait_scatter(b):
        pltpu.make_async_copy(rowss[b], acc.at[dstcs[b]], ssems[b]).wait()

    for b in range(NSLOT):
        start_chunk(b, b)

    def body(i, _):
        for b in range(NSLOT):
            j = NSLOT * i + b

            @pl.when(j < NCH)
            def _():
                wait_chunk(j, b)
                start_scatter(b)

            @pl.when(j + NSLOT < NCH)
            def _():
                wait_scatter(b)
                start_chunk(j + NSLOT, b)
        return 0
    lax.fori_loop(0, (NCH + NSLOT - 1) // NSLOT, body, 0)
    for b in range(NSLOT):
        wait_scatter(b)

    plsc.subcore_barrier()
    pltpu.sync_copy(acc.at[pl.ds(rbase, RPT)],
                    out_hbm.at[pl.ds(coff + rbase, RPT)])


_agg_call = pl.kernel(
    _agg_body,
    out_type=jax.ShapeDtypeStruct((NC * N, HALF), jnp.float32),
    mesh=_MESH,
    compiler_params=_SC_PARAMS,
    scratch_types=(
        [pltpu.VMEM((EPS,), jnp.int32)]
        + [pltpu.VMEM((CHUNK,), jnp.int32) for _ in range(NSLOT)]
        + [pltpu.VMEM((CHUNK, HALF), jnp.float32) for _ in range(NSLOT)]
        + [pltpu.VMEM_SHARED((N, HALF), jnp.float32)]
        + [pltpu.SemaphoreType.DMA for _ in range(1 + 3 * NSLOT)]
    ),
)


def _prep_body(x_ref, we_ref, be_ref, dp_ref, w1_ref, h_ref, dinv_ref, g_ref):
    h = jnp.dot(x_ref[...], we_ref[...],
                preferred_element_type=jnp.float32) + be_ref[...]
    h_ref[...] = h
    deg = jnp.sum(dp_ref[...], axis=(0, 1)) + 1.0
    dinv = lax.rsqrt(deg)[:, None]
    dinv_ref[...] = dinv
    t = jnp.dot(h, w1_ref[...], preferred_element_type=jnp.float32) * dinv
    g_ref[0] = t[:, :HALF]
    g_ref[1] = t[:, HALF:]


def _prep_call(x, w_emb, b_emb, degp, w1):
    return pl.pallas_call(
        _prep_body,
        out_shape=[
            jax.ShapeDtypeStruct((N, DH), jnp.float32),
            jax.ShapeDtypeStruct((N, 1), jnp.float32),
            jax.ShapeDtypeStruct((NC, N, HALF), jnp.float32),
        ],
    )(x, w_emb, b_emb, degp, w1)


def _fused_body(acc_ref, h_ref, dinv_ref, b_ref, w_ref, hn_ref, g_ref):
    a = jnp.concatenate([acc_ref[0], acc_ref[1]], axis=1)
    hn = jax.nn.relu(a * dinv_ref[...] + b_ref[...]) + h_ref[...]
    hn_ref[...] = hn
    t = jnp.dot(hn, w_ref[...],
                preferred_element_type=jnp.float32) * dinv_ref[...]
    g_ref[0] = t[:, :HALF]
    g_ref[1] = t[:, HALF:]


def _fused_call(acc, h, dinv, b, w):
    blk = 1000
    return pl.pallas_call(
        _fused_body,
        grid=(N // blk,),
        in_specs=[
            pl.BlockSpec((NC, blk, HALF), lambda i: (0, i, 0)),
            pl.BlockSpec((blk, DH), lambda i: (i, 0)),
            pl.BlockSpec((blk, 1), lambda i: (i, 0)),
            pl.BlockSpec((1, DH), lambda i: (0, 0)),
            pl.BlockSpec((DH, DH), lambda i: (0, 0)),
        ],
        out_specs=[
            pl.BlockSpec((blk, DH), lambda i: (i, 0)),
            pl.BlockSpec((NC, blk, HALF), lambda i: (0, i, 0)),
        ],
        out_shape=[
            jax.ShapeDtypeStruct((N, DH), jnp.float32),
            jax.ShapeDtypeStruct((NC, N, HALF), jnp.float32),
        ],
    )(acc, h, dinv, b, w)


def _post_body(acc_ref, h_ref, dinv_ref, b_ref, out_ref):
    a = jnp.concatenate([acc_ref[0], acc_ref[1]], axis=1)
    out_ref[...] = jax.nn.relu(a * dinv_ref[...] + b_ref[...]) + h_ref[...]


def _post_call(acc, h, dinv, b):
    blk = 1000
    return pl.pallas_call(
        _post_body,
        grid=(N // blk,),
        in_specs=[
            pl.BlockSpec((NC, blk, HALF), lambda i: (0, i, 0)),
            pl.BlockSpec((blk, DH), lambda i: (i, 0)),
            pl.BlockSpec((blk, 1), lambda i: (i, 0)),
            pl.BlockSpec((1, DH), lambda i: (0, 0)),
        ],
        out_specs=pl.BlockSpec((blk, DH), lambda i: (i, 0)),
        out_shape=jax.ShapeDtypeStruct((N, DH), jnp.float32),
    )(acc, h, dinv, b)


def kernel(x, edge_index, W_emb, b_emb, W1, b1, W2, b2, W3, b3):
    ei = edge_index.reshape(2 * E)
    degp = _deg_call(ei)
    h, dinv, g = _prep_call(x, W_emb, b_emb.reshape(1, DH), degp, W1)
    for b, W_next in ((b1, W2), (b2, W3)):
        accf = _agg_call(ei, g.reshape(NC * N, HALF))
        h, g = _fused_call(accf.reshape(NC, N, HALF), h, dinv,
                           b.reshape(1, DH), W_next)
    accf = _agg_call(ei, g.reshape(NC * N, HALF))
    return _post_call(accf.reshape(NC, N, HALF), h, dinv, b3.reshape(1, DH))
